# baseline (speedup 1.0000x reference)
import functools

import jax
import jax.numpy as jnp
from jax import lax
from jax.experimental import pallas as pl
from jax.experimental.pallas import tpu as pltpu
from jax.experimental.pallas import tpu_sc as plsc

NUM_CODES = 1024
DIM = 64
N_ROWS = 9216
NC, NS, LANES = 2, 16, 16
NW = NC * NS
B_PER_W = N_ROWS // NW
TC_BLOCK = 1152
IDX_W = 8


def _vq_idx_block(x_ref, e_ref, idx_ref, epad_ref):
    x = x_ref[...].reshape(TC_BLOCK, DIM)
    e = e_ref[...]
    en = jnp.sum(e * e, axis=1, keepdims=True)
    mmT = jax.lax.dot_general(
        e, x, (((1,), (1,)), ((), ())), preferred_element_type=jnp.float32)
    dT = en - 2.0 * mmT
    bits = jax.lax.bitcast_convert_type(dT, jnp.int32)
    key = jnp.where(bits < 0, jnp.int32(-2147483648) - bits, bits)
    code = jax.lax.broadcasted_iota(jnp.int32, dT.shape, 0)
    packed = (key & jnp.int32(~(NUM_CODES - 1))) | code
    m = packed[0:128, :]
    for k in range(1, 8):
        m = jnp.minimum(m, packed[128 * k:128 * (k + 1), :])
    m = jnp.min(m, axis=0, keepdims=True)
    idx = m & jnp.int32(NUM_CODES - 1)
    idx_ref[...] = jnp.broadcast_to(idx, (IDX_W, idx.shape[1]))
    i = pl.program_id(0)
    chunk = e_ref[pl.ds(i * 128, 128), :]
    epad_ref[...] = jnp.concatenate(
        [chunk, jnp.zeros((128, DIM), jnp.float32)], axis=1)


def _tc_indices(inputs, embedding):
    grid = (N_ROWS // TC_BLOCK,)
    return pl.pallas_call(
        _vq_idx_block,
        grid=grid,
        in_specs=[
            pl.BlockSpec((2, 576, DIM), lambda i: (i, 0, 0)),
            pl.BlockSpec((NUM_CODES, DIM), lambda i: (0, 0)),
        ],
        out_specs=[
            pl.BlockSpec((IDX_W, TC_BLOCK), lambda i: (0, i)),
            pl.BlockSpec((128, 2 * DIM), lambda i: (i, 0)),
        ],
        out_shape=[
            jax.ShapeDtypeStruct((IDX_W, N_ROWS), jnp.int32),
            jax.ShapeDtypeStruct((NUM_CODES, 2 * DIM), jnp.float32),
        ],
    )(inputs, embedding)


def _sc_gather_combine(e_hbm, idx_hbm, x_hbm, out_hbm,
                       rows_idx, rows_v, x_v, out_v, sem):
    wid = lax.axis_index("s") * NC + lax.axis_index("c")
    base = wid * B_PER_W
    pltpu.sync_copy(idx_hbm.at[pl.ds(base, B_PER_W)], rows_idx)
    pltpu.async_copy(e_hbm.at[rows_idx], rows_v, sem).wait()
    b = wid // 2
    h = wid % 2
    pltpu.sync_copy(x_hbm.at[b, pl.ds(h * B_PER_W, B_PER_W), :], x_v)

    @plsc.parallel_loop(0, B_PER_W, 1, unroll=8)
    def body(i):
        for j in range(DIM // LANES):
            sl = pl.ds(j * LANES, LANES)
            c = rows_v[i, sl]
            xx = x_v[i, sl]
            out_v[i, sl] = c + (xx - c)
    pltpu.sync_copy(out_v, out_hbm.at[b, pl.ds(h * B_PER_W, B_PER_W), :])


def kernel(inputs, embedding):
    idx, e_pad = _tc_indices(inputs, embedding)
    sc = pl.kernel(
        _sc_gather_combine,
        out_type=jax.ShapeDtypeStruct((16, 576, DIM), jnp.float32),
        mesh=plsc.VectorSubcoreMesh(core_axis_name="c", subcore_axis_name="s"),
        scratch_types=[
            pltpu.VMEM((B_PER_W,), jnp.int32),
            pltpu.VMEM((B_PER_W, 2 * DIM), jnp.float32),
            pltpu.VMEM((B_PER_W, DIM), jnp.float32),
            pltpu.VMEM((B_PER_W, DIM), jnp.float32),
            pltpu.SemaphoreType.DMA,
        ],
    )
    return sc(e_pad, idx.reshape(-1), inputs)


# speedup vs baseline: 1.0207x; 1.0207x over previous
import functools

import jax
import jax.numpy as jnp
from jax import lax
from jax.experimental import pallas as pl
from jax.experimental.pallas import tpu as pltpu
from jax.experimental.pallas import tpu_sc as plsc

NUM_CODES = 1024
DIM = 64
N_ROWS = 9216
NC, NS, LANES = 2, 16, 16
NW = NC * NS
B_PER_W = N_ROWS // NW
TC_BLOCK = 1152
IDX_W = 8


def _vq_idx_block(x_ref, e_ref, idx_ref, epad_ref):
    x = x_ref[...].reshape(TC_BLOCK, DIM)
    e = e_ref[...]
    en = jnp.sum(e * e, axis=1, keepdims=True)
    mmT = jax.lax.dot_general(
        e, x, (((1,), (1,)), ((), ())), preferred_element_type=jnp.float32)
    dT = en - 2.0 * mmT
    bits = jax.lax.bitcast_convert_type(dT, jnp.int32)
    key = jnp.where(bits < 0, jnp.int32(-2147483648) - bits, bits)
    code = jax.lax.broadcasted_iota(jnp.int32, dT.shape, 0)
    packed = (key & jnp.int32(~(NUM_CODES - 1))) | code
    m = packed[0:128, :]
    for k in range(1, 8):
        m = jnp.minimum(m, packed[128 * k:128 * (k + 1), :])
    m = jnp.min(m, axis=0, keepdims=True)
    idx = m & jnp.int32(NUM_CODES - 1)
    idx_ref[...] = jnp.broadcast_to(idx, (IDX_W, idx.shape[1]))
    i = pl.program_id(0)
    chunk = e_ref[pl.ds(i * 128, 128), :]
    epad_ref[...] = jnp.concatenate(
        [chunk, jnp.zeros((128, DIM), jnp.float32)], axis=1)


def _tc_indices(inputs, embedding):
    grid = (N_ROWS // TC_BLOCK,)
    return pl.pallas_call(
        _vq_idx_block,
        grid=grid,
        in_specs=[
            pl.BlockSpec((2, 576, DIM), lambda i: (i, 0, 0)),
            pl.BlockSpec((NUM_CODES, DIM), lambda i: (0, 0)),
        ],
        out_specs=[
            pl.BlockSpec((IDX_W, TC_BLOCK), lambda i: (0, i)),
            pl.BlockSpec((128, 2 * DIM), lambda i: (i, 0)),
        ],
        out_shape=[
            jax.ShapeDtypeStruct((IDX_W, N_ROWS), jnp.int32),
            jax.ShapeDtypeStruct((NUM_CODES, 2 * DIM), jnp.float32),
        ],
    )(inputs, embedding)


def _sc_gather_combine(e_hbm, idx_hbm, x_hbm, out_hbm,
                       rows_idx, rows_v, x_v, out_v, sem, sem2):
    wid = lax.axis_index("s") * NC + lax.axis_index("c")
    base = wid * B_PER_W
    b = wid // 2
    h = wid % 2
    x_copy = pltpu.async_copy(
        x_hbm.at[b, pl.ds(h * B_PER_W, B_PER_W), :], x_v, sem2)
    pltpu.sync_copy(idx_hbm.at[pl.ds(base, B_PER_W)], rows_idx)
    pltpu.async_copy(e_hbm.at[rows_idx], rows_v, sem).wait()
    x_copy.wait()

    @plsc.parallel_loop(0, B_PER_W, 1, unroll=8)
    def body(i):
        for j in range(DIM // LANES):
            sl = pl.ds(j * LANES, LANES)
            c = rows_v[i, sl]
            xx = x_v[i, sl]
            out_v[i, sl] = c + (xx - c)
    pltpu.sync_copy(out_v, out_hbm.at[b, pl.ds(h * B_PER_W, B_PER_W), :])


def kernel(inputs, embedding):
    idx, e_pad = _tc_indices(inputs, embedding)
    sc = pl.kernel(
        _sc_gather_combine,
        out_type=jax.ShapeDtypeStruct((16, 576, DIM), jnp.float32),
        mesh=plsc.VectorSubcoreMesh(core_axis_name="c", subcore_axis_name="s"),
        scratch_types=[
            pltpu.VMEM((B_PER_W,), jnp.int32),
            pltpu.VMEM((B_PER_W, 2 * DIM), jnp.float32),
            pltpu.VMEM((B_PER_W, DIM), jnp.float32),
            pltpu.VMEM((B_PER_W, DIM), jnp.float32),
            pltpu.SemaphoreType.DMA,
            pltpu.SemaphoreType.DMA,
        ],
    )
    return sc(e_pad, idx.reshape(-1), inputs)
